# baseline (speedup 1.0000x reference)
import functools

import jax
import jax.numpy as jnp
from jax import lax
from jax.experimental import pallas as pl
from jax.experimental.pallas import tpu as pltpu
from jax.experimental.pallas import tpu_sc as plsc

D = 128
B = 16384
NC, NS = 2, 16
NW = NC * NS
BPW = B // NW
CH = 64
K = BPW // CH

_mesh = plsc.VectorSubcoreMesh(core_axis_name="c", subcore_axis_name="s")


@functools.partial(
    pl.kernel,
    mesh=_mesh,
    out_type=jax.ShapeDtypeStruct((B, D), jnp.float32),
    scratch_types=[
        pltpu.VMEM((K, CH), jnp.int32),
        pltpu.VMEM((BPW, D), jnp.float32),
        pltpu.SemaphoreType.DMA((K,)),
        pltpu.SemaphoreType.DMA,
    ],
)
def _sc_gather(table_hbm, idx_hbm, out_hbm, idx_v, rows_v, gsems, wsem):
    wid = lax.axis_index("s") * NC + lax.axis_index("c")
    base = wid * BPW
    pltpu.sync_copy(idx_hbm.at[pl.ds(wid * K, K)], idx_v)

    def gather(j):
        return pltpu.async_copy(
            table_hbm.at[idx_v.at[j]], rows_v.at[pl.ds(j * CH, CH)], gsems.at[j]
        )

    def write(j):
        return pltpu.async_copy(
            rows_v.at[pl.ds(j * CH, CH)], out_hbm.at[pl.ds(base + j * CH, CH)], wsem
        )

    gathers = [gather(0), gather(1)]
    writes = []
    for j in range(K):
        gathers[j].wait()
        writes.append(write(j))
        if j + 2 < K:
            gathers.append(gather(j + 2))
    for w in writes:
        w.wait()


def kernel(electrode_emb, permutation, subject_id):
    idx2d = permutation.astype(jnp.int32).reshape(NW * K, CH)
    return _sc_gather(electrode_emb, idx2d)


# speedup vs baseline: 1.0251x; 1.0251x over previous
import functools

import jax
import jax.numpy as jnp
from jax import lax
from jax.experimental import pallas as pl
from jax.experimental.pallas import tpu as pltpu
from jax.experimental.pallas import tpu_sc as plsc

D = 128
B = 16384
NC, NS = 2, 16
NW = NC * NS
BPW = B // NW
CH = 128
K = BPW // CH

_mesh = plsc.VectorSubcoreMesh(core_axis_name="c", subcore_axis_name="s")


@functools.partial(
    pl.kernel,
    mesh=_mesh,
    out_type=jax.ShapeDtypeStruct((B, D), jnp.float32),
    scratch_types=[
        pltpu.VMEM((K, CH), jnp.int32),
        pltpu.VMEM((BPW, D), jnp.float32),
        pltpu.SemaphoreType.DMA((K,)),
        pltpu.SemaphoreType.DMA,
    ],
)
def _sc_gather(table_hbm, idx_hbm, out_hbm, idx_v, rows_v, gsems, wsem):
    wid = lax.axis_index("s") * NC + lax.axis_index("c")
    base = wid * BPW
    pltpu.sync_copy(idx_hbm.at[pl.ds(wid * K, K)], idx_v)

    def gather(j):
        return pltpu.async_copy(
            table_hbm.at[idx_v.at[j]], rows_v.at[pl.ds(j * CH, CH)], gsems.at[j]
        )

    def write(j):
        return pltpu.async_copy(
            rows_v.at[pl.ds(j * CH, CH)], out_hbm.at[pl.ds(base + j * CH, CH)], wsem
        )

    gathers = [gather(0), gather(1)]
    writes = []
    for j in range(K):
        gathers[j].wait()
        writes.append(write(j))
        if j + 2 < K:
            gathers.append(gather(j + 2))
    for w in writes:
        w.wait()


def kernel(electrode_emb, permutation, subject_id):
    idx2d = permutation.astype(jnp.int32).reshape(NW * K, CH)
    return _sc_gather(electrode_emb, idx2d)


# speedup vs baseline: 1.0753x; 1.0490x over previous
import functools

import jax
import jax.numpy as jnp
from jax import lax
from jax.experimental import pallas as pl
from jax.experimental.pallas import tpu as pltpu
from jax.experimental.pallas import tpu_sc as plsc

D = 128
B = 16384
NC, NS = 2, 16
NW = NC * NS
BPW = B // NW

_mesh = plsc.VectorSubcoreMesh(core_axis_name="c", subcore_axis_name="s")


@functools.partial(
    pl.kernel,
    mesh=_mesh,
    out_type=jax.ShapeDtypeStruct((B, D), jnp.float32),
    scratch_types=[
        pltpu.VMEM((BPW,), jnp.int32),
        pltpu.VMEM((BPW, D), jnp.float32),
        pltpu.SemaphoreType.DMA,
    ],
)
def _sc_gather(table_hbm, idx_hbm, out_hbm, idx_v, rows_v, sem):
    wid = lax.axis_index("s") * NC + lax.axis_index("c")
    base = wid * BPW
    pltpu.sync_copy(idx_hbm.at[pl.ds(base, BPW)], idx_v)
    pltpu.async_copy(table_hbm.at[idx_v], rows_v, sem).wait()
    pltpu.sync_copy(rows_v, out_hbm.at[pl.ds(base, BPW)])


def kernel(electrode_emb, permutation, subject_id):
    return _sc_gather(electrode_emb, permutation.astype(jnp.int32))
